# baseline (speedup 1.0000x reference)
import jax
import jax.numpy as jnp
from jax import lax
from jax.experimental import pallas as pl
from jax.experimental.pallas import tpu as pltpu
from jax.experimental.pallas import tpu_sc as plsc

_R = 1024
_NSRC = 32
_NB = 8
_H = 2048
_W = 2048


def _sc_rows_body(y2_hbm, idx_hbm, xt_hbm, syb_hbm, rows_out_hbm,
                  idx_v, xv_v, syv_v, rows_v, sem):
    wid = lax.axis_index("s") * 2 + lax.axis_index("c")
    pltpu.sync_copy(idx_hbm.at[wid], idx_v)
    pltpu.sync_copy(xt_hbm.at[wid], xv_v)
    pltpu.sync_copy(syb_hbm.at[wid], syv_v)
    pltpu.async_copy(y2_hbm.at[idx_v], rows_v, sem).wait()
    lanes = lax.broadcasted_iota(jnp.int32, (16,), 0)
    mask = lanes < _NB
    rowids = jnp.where(mask, lanes, 0)
    plsc.addupdate_scatter(rows_v, [rowids, syv_v[...]], xv_v[...], mask=mask)
    pltpu.sync_copy(rows_v, rows_out_hbm.at[wid])


def _sc_rows(Y, X, src_x, src_y):
    Y2 = Y.reshape(_NB * _H, _W)
    idx_all = (src_x[:, None] + _H * jnp.arange(_NB, dtype=jnp.int32)[None, :])
    xt16 = jnp.zeros((_NSRC, 16), jnp.float32).at[:, :_NB].set(X.T)
    syb = jnp.broadcast_to(src_y[:, None], (_NSRC, 16))
    mesh = plsc.VectorSubcoreMesh(core_axis_name="c", subcore_axis_name="s")
    f = pl.kernel(
        _sc_rows_body,
        out_type=jax.ShapeDtypeStruct((_NSRC, _NB, _W), jnp.float32),
        mesh=mesh,
        scratch_types=[
            pltpu.VMEM((_NB,), jnp.int32),
            pltpu.VMEM((16,), jnp.float32),
            pltpu.VMEM((16,), jnp.int32),
            pltpu.VMEM((_NB, _W), jnp.float32),
            pltpu.SemaphoreType.DMA,
        ],
        compiler_params=pltpu.CompilerParams(
            use_tc_tiling_on_sc=False, needs_layout_passes=False),
    )
    return f(Y2, idx_all, xt16, syb)


def _tc_body(src_x_ref, y_ref, rows_ref, out_ref):
    b = pl.program_id(0)
    rb = pl.program_id(1)
    r0 = rb * _R
    out_ref[...] = y_ref[...]
    for i in range(_NSRC):
        sx = src_x_ref[i]

        @pl.when(jnp.logical_and(sx >= r0, sx < r0 + _R))
        def _():
            xl = sx - r0
            out_ref[0, pl.ds(xl, 1), :] = rows_ref[i, pl.ds(b, 1), :]


def kernel(Y, X, src_x, src_y):
    rows = _sc_rows(Y, X, src_x, src_y)
    B, H, W = Y.shape
    grid = (B, H // _R)
    return pl.pallas_call(
        _tc_body,
        grid=grid,
        in_specs=[
            pl.BlockSpec(memory_space=pltpu.SMEM),
            pl.BlockSpec((1, _R, W), lambda b, r: (b, r, 0)),
            pl.BlockSpec((_NSRC, _NB, W), lambda b, r: (0, 0, 0)),
        ],
        out_specs=pl.BlockSpec((1, _R, W), lambda b, r: (b, r, 0)),
        out_shape=jax.ShapeDtypeStruct(Y.shape, Y.dtype),
        compiler_params=pltpu.CompilerParams(
            dimension_semantics=("parallel", "parallel"),
        ),
    )(src_x, Y, rows)


# speedup vs baseline: 2.4526x; 2.4526x over previous
import jax
import jax.numpy as jnp
from jax import lax
from jax.experimental import pallas as pl
from jax.experimental.pallas import tpu as pltpu

_R = 1024
_NSRC = 32


def _body(src_x_ref, src_y_ref, x_ref, y_ref, out_ref):
    b = pl.program_id(0)
    rb = pl.program_id(1)
    r0 = rb * _R
    out_ref[...] = y_ref[...]
    col = lax.broadcasted_iota(jnp.int32, (1, 2048), 1)
    for i in range(_NSRC):
        sx = src_x_ref[i]
        sy = src_y_ref[i]

        @pl.when(jnp.logical_and(sx >= r0, sx < r0 + _R))
        def _():
            xl = sx - r0
            xv = x_ref[b, i]
            row = out_ref[0, pl.ds(xl, 1), :]
            out_ref[0, pl.ds(xl, 1), :] = row + jnp.where(col == sy, xv, 0.0)


def kernel(Y, X, src_x, src_y):
    B, H, W = Y.shape
    grid = (B, H // _R)
    return pl.pallas_call(
        _body,
        grid=grid,
        in_specs=[
            pl.BlockSpec(memory_space=pltpu.SMEM),
            pl.BlockSpec(memory_space=pltpu.SMEM),
            pl.BlockSpec(memory_space=pltpu.SMEM),
            pl.BlockSpec((1, _R, W), lambda b, r: (b, r, 0)),
        ],
        out_specs=pl.BlockSpec((1, _R, W), lambda b, r: (b, r, 0)),
        out_shape=jax.ShapeDtypeStruct(Y.shape, Y.dtype),
        compiler_params=pltpu.CompilerParams(
            dimension_semantics=("parallel", "parallel"),
        ),
    )(src_x, src_y, X, Y)
